# baseline (speedup 1.0000x reference)
import functools

import jax
import jax.numpy as jnp
from jax import lax
from jax.experimental import pallas as pl
from jax.experimental.pallas import tpu as pltpu
from jax.experimental.pallas import tpu_sc as plsc

_SQRT_HALF = 0.7071067811865476


def _gate_body(x_ref, Wg_ref, bg_ref, idx0_ref, idx1_ref, rank0_ref,
               rank1_ref, w0_ref, w1_ref, cnt_ref, off_ref, aux_ref,
               base_ref, ent_ref, *, NB, E, N):
    i = pl.program_id(0)
    s = jnp.dot(x_ref[...], Wg_ref[...],
                preferred_element_type=jnp.float32) + bg_ref[...]
    BN = s.shape[0]
    iota = lax.broadcasted_iota(jnp.int32, s.shape, 1)
    m1 = jnp.max(s, axis=1, keepdims=True)
    i1 = jnp.min(jnp.where(s == m1, iota, E), axis=1, keepdims=True)
    s2 = jnp.where(iota == i1, -jnp.inf, s)
    m2 = jnp.max(s2, axis=1, keepdims=True)
    i2 = jnp.min(jnp.where(s2 == m2, iota, E), axis=1, keepdims=True)
    w0 = jax.nn.sigmoid(m1 - m2)
    oh0 = (iota == i1).astype(jnp.float32)
    oh1 = (iota == i2).astype(jnp.float32)
    pairsum = oh0 + oh1

    @pl.when(i == 0)
    def _init():
        base_ref[...] = jnp.zeros_like(base_ref)
        ent_ref[0, 0] = 0.0

    rowi = lax.broadcasted_iota(jnp.int32, (BN, BN), 0)
    coli = lax.broadcasted_iota(jnp.int32, (BN, BN), 1)
    tri = (coli < rowi).astype(jnp.float32)
    prev = jnp.dot(tri, pairsum, preferred_element_type=jnp.float32)
    base_plus = base_ref[...] + prev
    rank0 = jnp.sum(oh0 * base_plus, axis=1, keepdims=True)
    rank1 = jnp.sum(oh1 * base_plus, axis=1, keepdims=True)

    idx0_ref[...] = i1
    idx1_ref[...] = i2
    rank0_ref[...] = rank0.astype(jnp.int32)
    rank1_ref[...] = rank1.astype(jnp.int32)
    w0_ref[...] = w0
    w1_ref[...] = 1.0 - w0

    base_ref[...] += jnp.sum(pairsum, axis=0, keepdims=True)
    lse = m1 + jnp.log(jnp.sum(jnp.exp(s - m1), axis=1, keepdims=True))
    logp = s - lse
    ent_ref[0, 0] += -jnp.sum(jnp.exp(logp) * logp)

    @pl.when(i == NB - 1)
    def _finalize():
        cnt_ref[...] = base_ref[...].astype(jnp.int32)
        er = lax.broadcasted_iota(jnp.int32, (E, E), 0)
        ec = lax.broadcasted_iota(jnp.int32, (E, E), 1)
        triu = (er < ec).astype(jnp.float32)
        off_ref[...] = jnp.dot(base_ref[...], triu,
                               preferred_element_type=jnp.float32,
                               precision=lax.Precision.HIGHEST
                               ).astype(jnp.int32)
        usage = base_ref[...] / N
        lb = jnp.mean((usage - 1.0 / E) ** 2)
        aux_ref[0, 0] = lb - 0.1 * (ent_ref[0, 0] / N)


def _run_gate(x, Wg, bg, *, BN):
    N, D = x.shape
    E = Wg.shape[1]
    NB = N // BN
    return pl.pallas_call(
        functools.partial(_gate_body, NB=NB, E=E, N=N),
        grid=(NB,),
        in_specs=[
            pl.BlockSpec((BN, D), lambda i: (i, 0)),
            pl.BlockSpec((D, E), lambda i: (0, 0)),
            pl.BlockSpec((1, E), lambda i: (0, 0)),
        ],
        out_specs=[pl.BlockSpec((BN, 1), lambda i: (i, 0))] * 6 + [
            pl.BlockSpec((1, E), lambda i: (0, 0)),
            pl.BlockSpec((1, E), lambda i: (0, 0)),
            pl.BlockSpec(memory_space=pltpu.SMEM),
        ],
        out_shape=[
            jax.ShapeDtypeStruct((N, 1), jnp.int32),
            jax.ShapeDtypeStruct((N, 1), jnp.int32),
            jax.ShapeDtypeStruct((N, 1), jnp.int32),
            jax.ShapeDtypeStruct((N, 1), jnp.int32),
            jax.ShapeDtypeStruct((N, 1), jnp.float32),
            jax.ShapeDtypeStruct((N, 1), jnp.float32),
            jax.ShapeDtypeStruct((1, E), jnp.int32),
            jax.ShapeDtypeStruct((1, E), jnp.int32),
            jax.ShapeDtypeStruct((1, 1), jnp.float32),
        ],
        scratch_shapes=[
            pltpu.VMEM((1, E), jnp.float32),
            pltpu.SMEM((1, 1), jnp.float32),
        ],
    )(x, Wg, bg.reshape(1, E))


def _sc_scatter(x, idx0, idx1, rank0, rank1, off):
    N, D = x.shape
    E = off.shape[0]
    M = 2 * N
    info = plsc.get_sparse_core_info()
    NC, NS = info.num_cores, info.num_subcores
    NW = NC * NS
    TPW = N // NW
    TB = 32
    mesh = plsc.VectorSubcoreMesh(core_axis_name="c", subcore_axis_name="s")

    @functools.partial(
        pl.kernel, mesh=mesh,
        compiler_params=pltpu.CompilerParams(needs_layout_passes=False),
        out_type=[
            jax.ShapeDtypeStruct((M, D), jnp.float32),
            jax.ShapeDtypeStruct((N,), jnp.int32),
            jax.ShapeDtypeStruct((N,), jnp.int32),
        ],
        scratch_types=[
            pltpu.VMEM((E,), jnp.int32),
            pltpu.VMEM((TB,), jnp.int32),
            pltpu.VMEM((TB,), jnp.int32),
            pltpu.VMEM((TB,), jnp.int32),
            pltpu.VMEM((TB,), jnp.int32),
            pltpu.VMEM((TB,), jnp.int32),
            pltpu.VMEM((TB,), jnp.int32),
            pltpu.VMEM((TB, D), jnp.float32),
            pltpu.SemaphoreType.DMA,
            pltpu.SemaphoreType.DMA,
        ],
    )
    def k(x_hbm, i0_hbm, i1_hbm, r0_hbm, r1_hbm, off_hbm,
          xs_hbm, d0_hbm, d1_hbm,
          off_v, i0v, i1v, r0v, r1v, d0v, d1v, xbuf, sem0, sem1):
        wid = lax.axis_index("s") * NC + lax.axis_index("c")
        pltpu.sync_copy(off_hbm, off_v)
        base = wid * TPW

        def chunk(c, carry):
            t0 = base + c * TB
            pltpu.sync_copy(i0_hbm.at[pl.ds(t0, TB)], i0v)
            pltpu.sync_copy(i1_hbm.at[pl.ds(t0, TB)], i1v)
            pltpu.sync_copy(r0_hbm.at[pl.ds(t0, TB)], r0v)
            pltpu.sync_copy(r1_hbm.at[pl.ds(t0, TB)], r1v)
            for j in range(TB // 16):
                sl = pl.ds(j * 16, 16)
                d0v[sl] = plsc.load_gather(off_v, [i0v[sl]]) + r0v[sl]
                d1v[sl] = plsc.load_gather(off_v, [i1v[sl]]) + r1v[sl]
            pltpu.sync_copy(d0v, d0_hbm.at[pl.ds(t0, TB)])
            pltpu.sync_copy(d1v, d1_hbm.at[pl.ds(t0, TB)])
            pltpu.sync_copy(x_hbm.at[pl.ds(t0, TB)], xbuf)
            cp0 = pltpu.async_copy(xbuf, xs_hbm.at[d0v], sem0)
            cp1 = pltpu.async_copy(xbuf, xs_hbm.at[d1v], sem1)
            cp0.wait()
            cp1.wait()
            return carry

        lax.fori_loop(0, TPW // TB, chunk, 0)

    return k(x, idx0, idx1, rank0, rank1, off)


def _group_body(s_tile, s_exp, s_rows, xs_ref, W1_ref, b1_ref, g_ref,
                be_ref, W2_ref, b2_ref, ys_ref):
    i = pl.program_id(0)
    rs = s_rows[0, i]
    re_ = s_rows[1, i]

    @pl.when(re_ > rs)
    def _work():
        h = jnp.dot(xs_ref[...].astype(jnp.bfloat16), W1_ref[0],
                    preferred_element_type=jnp.float32) + b1_ref[0]
        h = 0.5 * h * (1.0 + lax.erf(h * jnp.float32(_SQRT_HALF)))
        mu = jnp.mean(h, axis=-1, keepdims=True)
        var = jnp.mean((h - mu) ** 2, axis=-1, keepdims=True)
        hn = (h - mu) / jnp.sqrt(var + 1e-5) * g_ref[0] + be_ref[0]
        y = jnp.dot(hn.astype(jnp.bfloat16), W2_ref[0],
                    preferred_element_type=jnp.float32) + b2_ref[0]
        ri = lax.broadcasted_iota(jnp.int32, y.shape, 0)
        ys_ref[...] = jnp.where((ri >= rs) & (ri < re_), y, ys_ref[...])


def _run_group(xs, W1, b1, gamma, beta, W2, b2, cnt, *, BT):
    M, D = xs.shape
    E, _, H = W1.shape
    O = W2.shape[2]
    NT = M // BT
    NW = NT + E - 1

    cnt = cnt.astype(jnp.int32)
    off = jnp.concatenate(
        [jnp.zeros((1,), jnp.int32), jnp.cumsum(cnt)[:-1].astype(jnp.int32)])
    tile_pts = jnp.arange(NT + 1, dtype=jnp.int32) * BT
    pts = jnp.sort(jnp.concatenate([tile_pts, off[1:]]))
    a, b = pts[:-1], pts[1:]
    s_tile = jnp.minimum(a // BT, NT - 1).astype(jnp.int32)
    s_exp = jnp.clip(jnp.searchsorted(off, a, side="right").astype(jnp.int32)
                     - 1, 0, E - 1)
    rs = a - s_tile * BT
    re_ = jnp.maximum(b - s_tile * BT, rs)
    s_rows = jnp.stack([rs, re_]).astype(jnp.int32)

    grid_spec = pltpu.PrefetchScalarGridSpec(
        num_scalar_prefetch=3,
        grid=(NW,),
        in_specs=[
            pl.BlockSpec((BT, D), lambda i, st, se, sr: (st[i], 0)),
            pl.BlockSpec((1, D, H), lambda i, st, se, sr: (se[i], 0, 0)),
            pl.BlockSpec((1, 1, H), lambda i, st, se, sr: (se[i], 0, 0)),
            pl.BlockSpec((1, 1, H), lambda i, st, se, sr: (se[i], 0, 0)),
            pl.BlockSpec((1, 1, H), lambda i, st, se, sr: (se[i], 0, 0)),
            pl.BlockSpec((1, H, O), lambda i, st, se, sr: (se[i], 0, 0)),
            pl.BlockSpec((1, 1, O), lambda i, st, se, sr: (se[i], 0, 0)),
        ],
        out_specs=pl.BlockSpec((BT, O), lambda i, st, se, sr: (st[i], 0)),
    )
    return pl.pallas_call(
        _group_body,
        grid_spec=grid_spec,
        out_shape=jax.ShapeDtypeStruct((M, O), jnp.float32),
    )(s_tile, s_exp, s_rows, xs, W1.astype(jnp.bfloat16),
      b1.reshape(E, 1, H), gamma.reshape(E, 1, H), beta.reshape(E, 1, H),
      W2.astype(jnp.bfloat16), b2.reshape(E, 1, O))


def _sc_combine(ys, d0, d1, w0, w1):
    M, O = ys.shape
    N = d0.shape[0]
    info = plsc.get_sparse_core_info()
    NC, NS = info.num_cores, info.num_subcores
    NW = NC * NS
    TPW = N // NW
    TC = 64
    mesh = plsc.VectorSubcoreMesh(core_axis_name="c", subcore_axis_name="s")

    @functools.partial(
        pl.kernel, mesh=mesh,
        compiler_params=pltpu.CompilerParams(needs_layout_passes=False),
        out_type=jax.ShapeDtypeStruct((N, O), jnp.float32),
        scratch_types=[
            pltpu.VMEM((TC,), jnp.int32),
            pltpu.VMEM((TC,), jnp.int32),
            pltpu.VMEM((TC,), jnp.float32),
            pltpu.VMEM((TC,), jnp.float32),
            pltpu.VMEM((TC, O), jnp.float32),
            pltpu.VMEM((TC, O), jnp.float32),
            pltpu.VMEM((TC, O), jnp.float32),
            pltpu.SemaphoreType.DMA,
            pltpu.SemaphoreType.DMA,
        ],
    )
    def k(ys_hbm, d0_hbm, d1_hbm, w0_hbm, w1_hbm, out_hbm,
          i0v, i1v, w0v, w1v, b0, b1, ob, sem0, sem1):
        wid = lax.axis_index("s") * NC + lax.axis_index("c")
        base = wid * TPW

        def chunk(c, carry):
            t0 = base + c * TC
            pltpu.sync_copy(d0_hbm.at[pl.ds(t0, TC)], i0v)
            pltpu.sync_copy(d1_hbm.at[pl.ds(t0, TC)], i1v)
            pltpu.sync_copy(w0_hbm.at[pl.ds(t0, TC)], w0v)
            pltpu.sync_copy(w1_hbm.at[pl.ds(t0, TC)], w1v)
            g0 = pltpu.async_copy(ys_hbm.at[i0v], b0, sem0)
            g1 = pltpu.async_copy(ys_hbm.at[i1v], b1, sem1)
            g0.wait()
            g1.wait()

            def token(n, tc):
                nv = jnp.full((16,), n, jnp.int32)
                wv0 = plsc.load_gather(w0v, [nv])
                wv1 = plsc.load_gather(w1v, [nv])
                for j in range(O // 16):
                    sl = pl.ds(j * 16, 16)
                    ob[n, sl] = b0[n, sl] * wv0 + b1[n, sl] * wv1
                return tc

            lax.fori_loop(0, TC, token, 0)
            pltpu.sync_copy(ob, out_hbm.at[pl.ds(t0, TC)])
            return carry

        lax.fori_loop(0, TPW // TC, chunk, 0)

    return k(ys, d0, d1, w0, w1)


@jax.jit
def kernel(x, Wg, bg, W1, b1, gamma, beta, W2, b2):
    N, D = x.shape
    E = Wg.shape[1]

    (idx0, idx1, rank0, rank1, w0, w1, cnt, off, aux) = _run_gate(
        x, Wg, bg, BN=256)

    cnt16 = cnt.reshape(E)
    xs, d0, d1 = _sc_scatter(
        x, idx0.reshape(N), idx1.reshape(N), rank0.reshape(N),
        rank1.reshape(N), off.reshape(E))

    ys = _run_group(xs, W1, b1, gamma, beta, W2, b2, cnt16, BT=512)

    out = _sc_combine(ys, d0, d1, w0.reshape(N), w1.reshape(N))
    return out, aux[0, 0]


# speedup vs baseline: 1.0603x; 1.0603x over previous
import functools

import jax
import jax.numpy as jnp
from jax import lax
from jax.experimental import pallas as pl
from jax.experimental.pallas import tpu as pltpu
from jax.experimental.pallas import tpu_sc as plsc

_SQRT_HALF = 0.7071067811865476


def _gate_body(x_ref, Wg_ref, bg_ref, idx0_ref, idx1_ref, rank0_ref,
               rank1_ref, w0_ref, w1_ref, cnt_ref, off_ref, aux_ref,
               base_ref, ent_ref, *, NB, E, N):
    i = pl.program_id(0)
    s = jnp.dot(x_ref[...], Wg_ref[...],
                preferred_element_type=jnp.float32) + bg_ref[...]
    BN = s.shape[0]
    iota = lax.broadcasted_iota(jnp.int32, s.shape, 1)
    m1 = jnp.max(s, axis=1, keepdims=True)
    i1 = jnp.min(jnp.where(s == m1, iota, E), axis=1, keepdims=True)
    s2 = jnp.where(iota == i1, -jnp.inf, s)
    m2 = jnp.max(s2, axis=1, keepdims=True)
    i2 = jnp.min(jnp.where(s2 == m2, iota, E), axis=1, keepdims=True)
    w0 = jax.nn.sigmoid(m1 - m2)
    oh0 = (iota == i1).astype(jnp.float32)
    oh1 = (iota == i2).astype(jnp.float32)
    pairsum = oh0 + oh1

    @pl.when(i == 0)
    def _init():
        base_ref[...] = jnp.zeros_like(base_ref)
        ent_ref[0, 0] = 0.0

    rowi = lax.broadcasted_iota(jnp.int32, (BN, BN), 0)
    coli = lax.broadcasted_iota(jnp.int32, (BN, BN), 1)
    tri = (coli < rowi).astype(jnp.float32)
    prev = jnp.dot(tri, pairsum, preferred_element_type=jnp.float32)
    base_plus = base_ref[...] + prev
    rank0 = jnp.sum(oh0 * base_plus, axis=1, keepdims=True)
    rank1 = jnp.sum(oh1 * base_plus, axis=1, keepdims=True)

    idx0_ref[...] = i1
    idx1_ref[...] = i2
    rank0_ref[...] = rank0.astype(jnp.int32)
    rank1_ref[...] = rank1.astype(jnp.int32)
    w0_ref[...] = w0
    w1_ref[...] = 1.0 - w0

    base_ref[...] += jnp.sum(pairsum, axis=0, keepdims=True)
    lse = m1 + jnp.log(jnp.sum(jnp.exp(s - m1), axis=1, keepdims=True))
    logp = s - lse
    ent_ref[0, 0] += -jnp.sum(jnp.exp(logp) * logp)

    @pl.when(i == NB - 1)
    def _finalize():
        cnt_ref[...] = base_ref[...].astype(jnp.int32)
        er = lax.broadcasted_iota(jnp.int32, (E, E), 0)
        ec = lax.broadcasted_iota(jnp.int32, (E, E), 1)
        triu = (er < ec).astype(jnp.float32)
        off_ref[...] = jnp.dot(base_ref[...], triu,
                               preferred_element_type=jnp.float32,
                               precision=lax.Precision.HIGHEST
                               ).astype(jnp.int32)
        usage = base_ref[...] / N
        lb = jnp.mean((usage - 1.0 / E) ** 2)
        aux_ref[0, 0] = lb - 0.1 * (ent_ref[0, 0] / N)


def _run_gate(x, Wg, bg, *, BN):
    N, D = x.shape
    E = Wg.shape[1]
    NB = N // BN
    return pl.pallas_call(
        functools.partial(_gate_body, NB=NB, E=E, N=N),
        grid=(NB,),
        in_specs=[
            pl.BlockSpec((BN, D), lambda i: (i, 0)),
            pl.BlockSpec((D, E), lambda i: (0, 0)),
            pl.BlockSpec((1, E), lambda i: (0, 0)),
        ],
        out_specs=[pl.BlockSpec((BN, 1), lambda i: (i, 0))] * 6 + [
            pl.BlockSpec((1, E), lambda i: (0, 0)),
            pl.BlockSpec((1, E), lambda i: (0, 0)),
            pl.BlockSpec(memory_space=pltpu.SMEM),
        ],
        out_shape=[
            jax.ShapeDtypeStruct((N, 1), jnp.int32),
            jax.ShapeDtypeStruct((N, 1), jnp.int32),
            jax.ShapeDtypeStruct((N, 1), jnp.int32),
            jax.ShapeDtypeStruct((N, 1), jnp.int32),
            jax.ShapeDtypeStruct((N, 1), jnp.float32),
            jax.ShapeDtypeStruct((N, 1), jnp.float32),
            jax.ShapeDtypeStruct((1, E), jnp.int32),
            jax.ShapeDtypeStruct((1, E), jnp.int32),
            jax.ShapeDtypeStruct((1, 1), jnp.float32),
        ],
        scratch_shapes=[
            pltpu.VMEM((1, E), jnp.float32),
            pltpu.SMEM((1, 1), jnp.float32),
        ],
    )(x, Wg, bg.reshape(1, E))


def _sc_scatter(x, idx0, idx1, rank0, rank1, off):
    N, D = x.shape
    E = off.shape[0]
    M = 2 * N
    info = plsc.get_sparse_core_info()
    NC, NS = info.num_cores, info.num_subcores
    NW = NC * NS
    TPW = N // NW
    TB = 32
    mesh = plsc.VectorSubcoreMesh(core_axis_name="c", subcore_axis_name="s")

    @functools.partial(
        pl.kernel, mesh=mesh,
        compiler_params=pltpu.CompilerParams(needs_layout_passes=False),
        out_type=[
            jax.ShapeDtypeStruct((M, D), jnp.float32),
            jax.ShapeDtypeStruct((N,), jnp.int32),
            jax.ShapeDtypeStruct((N,), jnp.int32),
        ],
        scratch_types=[
            pltpu.VMEM((E,), jnp.int32),
            pltpu.VMEM((TB,), jnp.int32),
            pltpu.VMEM((TB,), jnp.int32),
            pltpu.VMEM((TB,), jnp.int32),
            pltpu.VMEM((TB,), jnp.int32),
            pltpu.VMEM((TB,), jnp.int32),
            pltpu.VMEM((TB,), jnp.int32),
            pltpu.VMEM((TB, D), jnp.float32),
            pltpu.SemaphoreType.DMA,
            pltpu.SemaphoreType.DMA,
        ],
    )
    def k(x_hbm, i0_hbm, i1_hbm, r0_hbm, r1_hbm, off_hbm,
          xs_hbm, d0_hbm, d1_hbm,
          off_v, i0v, i1v, r0v, r1v, d0v, d1v, xbuf, sem0, sem1):
        wid = lax.axis_index("s") * NC + lax.axis_index("c")
        pltpu.sync_copy(off_hbm, off_v)
        base = wid * TPW

        def chunk(c, carry):
            t0 = base + c * TB
            pltpu.sync_copy(i0_hbm.at[pl.ds(t0, TB)], i0v)
            pltpu.sync_copy(i1_hbm.at[pl.ds(t0, TB)], i1v)
            pltpu.sync_copy(r0_hbm.at[pl.ds(t0, TB)], r0v)
            pltpu.sync_copy(r1_hbm.at[pl.ds(t0, TB)], r1v)
            for j in range(TB // 16):
                sl = pl.ds(j * 16, 16)
                d0v[sl] = plsc.load_gather(off_v, [i0v[sl]]) + r0v[sl]
                d1v[sl] = plsc.load_gather(off_v, [i1v[sl]]) + r1v[sl]
            pltpu.sync_copy(d0v, d0_hbm.at[pl.ds(t0, TB)])
            pltpu.sync_copy(d1v, d1_hbm.at[pl.ds(t0, TB)])
            pltpu.sync_copy(x_hbm.at[pl.ds(t0, TB)], xbuf)
            cp0 = pltpu.async_copy(xbuf, xs_hbm.at[d0v], sem0)
            cp1 = pltpu.async_copy(xbuf, xs_hbm.at[d1v], sem1)
            cp0.wait()
            cp1.wait()
            return carry

        lax.fori_loop(0, TPW // TB, chunk, 0)

    return k(x, idx0, idx1, rank0, rank1, off)


def _group_body(s_tile, s_exp, s_rows, xs_ref, W1_ref, b1_ref, g_ref,
                be_ref, W2_ref, b2_ref, ys_ref):
    i = pl.program_id(0)
    rs = s_rows[0, i]
    re_ = s_rows[1, i]

    @pl.when(re_ > rs)
    def _work():
        h = jnp.dot(xs_ref[...].astype(jnp.bfloat16),
                    W1_ref[0].astype(jnp.bfloat16),
                    preferred_element_type=jnp.float32) + b1_ref[0]
        h = 0.5 * h * (1.0 + lax.erf(h * jnp.float32(_SQRT_HALF)))
        mu = jnp.mean(h, axis=-1, keepdims=True)
        var = jnp.mean((h - mu) ** 2, axis=-1, keepdims=True)
        hn = (h - mu) / jnp.sqrt(var + 1e-5) * g_ref[0] + be_ref[0]
        y = jnp.dot(hn.astype(jnp.bfloat16), W2_ref[0].astype(jnp.bfloat16),
                    preferred_element_type=jnp.float32) + b2_ref[0]
        ri = lax.broadcasted_iota(jnp.int32, y.shape, 0)
        ys_ref[...] = jnp.where((ri >= rs) & (ri < re_), y, ys_ref[...])


def _run_group(xs, W1, b1, gamma, beta, W2, b2, cnt, *, BT):
    M, D = xs.shape
    E, _, H = W1.shape
    O = W2.shape[2]
    NT = M // BT
    NW = NT + E - 1

    cnt = cnt.astype(jnp.int32)
    off = jnp.concatenate(
        [jnp.zeros((1,), jnp.int32), jnp.cumsum(cnt)[:-1].astype(jnp.int32)])
    tile_pts = jnp.arange(NT + 1, dtype=jnp.int32) * BT
    pts = jnp.sort(jnp.concatenate([tile_pts, off[1:]]))
    a, b = pts[:-1], pts[1:]
    s_tile = jnp.minimum(a // BT, NT - 1).astype(jnp.int32)
    s_exp = jnp.clip(jnp.searchsorted(off, a, side="right").astype(jnp.int32)
                     - 1, 0, E - 1)
    rs = a - s_tile * BT
    re_ = jnp.maximum(b - s_tile * BT, rs)
    s_rows = jnp.stack([rs, re_]).astype(jnp.int32)

    grid_spec = pltpu.PrefetchScalarGridSpec(
        num_scalar_prefetch=3,
        grid=(NW,),
        in_specs=[
            pl.BlockSpec((BT, D), lambda i, st, se, sr: (st[i], 0)),
            pl.BlockSpec((1, D, H), lambda i, st, se, sr: (se[i], 0, 0)),
            pl.BlockSpec((1, 1, H), lambda i, st, se, sr: (se[i], 0, 0)),
            pl.BlockSpec((1, 1, H), lambda i, st, se, sr: (se[i], 0, 0)),
            pl.BlockSpec((1, 1, H), lambda i, st, se, sr: (se[i], 0, 0)),
            pl.BlockSpec((1, H, O), lambda i, st, se, sr: (se[i], 0, 0)),
            pl.BlockSpec((1, 1, O), lambda i, st, se, sr: (se[i], 0, 0)),
        ],
        out_specs=pl.BlockSpec((BT, O), lambda i, st, se, sr: (st[i], 0)),
    )
    return pl.pallas_call(
        _group_body,
        grid_spec=grid_spec,
        out_shape=jax.ShapeDtypeStruct((M, O), jnp.float32),
    )(s_tile, s_exp, s_rows, xs, W1,
      b1.reshape(E, 1, H), gamma.reshape(E, 1, H), beta.reshape(E, 1, H),
      W2, b2.reshape(E, 1, O))


def _sc_combine(ys, d0, d1, w0, w1):
    M, O = ys.shape
    N = d0.shape[0]
    info = plsc.get_sparse_core_info()
    NC, NS = info.num_cores, info.num_subcores
    NW = NC * NS
    TPW = N // NW
    TC = 64
    mesh = plsc.VectorSubcoreMesh(core_axis_name="c", subcore_axis_name="s")

    @functools.partial(
        pl.kernel, mesh=mesh,
        compiler_params=pltpu.CompilerParams(needs_layout_passes=False),
        out_type=jax.ShapeDtypeStruct((N, O), jnp.float32),
        scratch_types=[
            pltpu.VMEM((TC,), jnp.int32),
            pltpu.VMEM((TC,), jnp.int32),
            pltpu.VMEM((TC,), jnp.float32),
            pltpu.VMEM((TC,), jnp.float32),
            pltpu.VMEM((TC, O), jnp.float32),
            pltpu.VMEM((TC, O), jnp.float32),
            pltpu.VMEM((TC, O), jnp.float32),
            pltpu.SemaphoreType.DMA,
            pltpu.SemaphoreType.DMA,
        ],
    )
    def k(ys_hbm, d0_hbm, d1_hbm, w0_hbm, w1_hbm, out_hbm,
          i0v, i1v, w0v, w1v, b0, b1, ob, sem0, sem1):
        wid = lax.axis_index("s") * NC + lax.axis_index("c")
        base = wid * TPW

        def chunk(c, carry):
            t0 = base + c * TC
            pltpu.sync_copy(d0_hbm.at[pl.ds(t0, TC)], i0v)
            pltpu.sync_copy(d1_hbm.at[pl.ds(t0, TC)], i1v)
            pltpu.sync_copy(w0_hbm.at[pl.ds(t0, TC)], w0v)
            pltpu.sync_copy(w1_hbm.at[pl.ds(t0, TC)], w1v)
            g0 = pltpu.async_copy(ys_hbm.at[i0v], b0, sem0)
            g1 = pltpu.async_copy(ys_hbm.at[i1v], b1, sem1)
            g0.wait()
            g1.wait()

            def token(n, tc):
                nv = jnp.full((16,), n, jnp.int32)
                wv0 = plsc.load_gather(w0v, [nv])
                wv1 = plsc.load_gather(w1v, [nv])
                for j in range(O // 16):
                    sl = pl.ds(j * 16, 16)
                    ob[n, sl] = b0[n, sl] * wv0 + b1[n, sl] * wv1
                return tc

            lax.fori_loop(0, TC, token, 0)
            pltpu.sync_copy(ob, out_hbm.at[pl.ds(t0, TC)])
            return carry

        lax.fori_loop(0, TPW // TC, chunk, 0)

    return k(ys, d0, d1, w0, w1)


@jax.jit
def kernel(x, Wg, bg, W1, b1, gamma, beta, W2, b2):
    N, D = x.shape
    E = Wg.shape[1]

    (idx0, idx1, rank0, rank1, w0, w1, cnt, off, aux) = _run_gate(
        x, Wg, bg, BN=512)

    cnt16 = cnt.reshape(E)
    xs, d0, d1 = _sc_scatter(
        x, idx0.reshape(N), idx1.reshape(N), rank0.reshape(N),
        rank1.reshape(N), off.reshape(E))

    ys = _run_group(xs, W1, b1, gamma, beta, W2, b2, cnt16, BT=1024)

    out = _sc_combine(ys, d0, d1, w0.reshape(N), w1.reshape(N))
    return out, aux[0, 0]


# speedup vs baseline: 1.0844x; 1.0227x over previous
import functools

import jax
import jax.numpy as jnp
from jax import lax
from jax.experimental import pallas as pl
from jax.experimental.pallas import tpu as pltpu
from jax.experimental.pallas import tpu_sc as plsc

_SQRT_HALF = 0.7071067811865476


def _gate_body(x_ref, Wg_ref, bg_ref, idx0_ref, idx1_ref, rank0_ref,
               rank1_ref, w0_ref, w1_ref, cnt_ref, off_ref, aux_ref,
               base_ref, ent_ref, *, NB, E, N):
    i = pl.program_id(0)
    s = jnp.dot(x_ref[...], Wg_ref[...],
                preferred_element_type=jnp.float32) + bg_ref[...]
    BN = s.shape[0]
    iota = lax.broadcasted_iota(jnp.int32, s.shape, 1)
    m1 = jnp.max(s, axis=1, keepdims=True)
    i1 = jnp.min(jnp.where(s == m1, iota, E), axis=1, keepdims=True)
    s2 = jnp.where(iota == i1, -jnp.inf, s)
    m2 = jnp.max(s2, axis=1, keepdims=True)
    i2 = jnp.min(jnp.where(s2 == m2, iota, E), axis=1, keepdims=True)
    w0 = jax.nn.sigmoid(m1 - m2)
    oh0 = (iota == i1).astype(jnp.float32)
    oh1 = (iota == i2).astype(jnp.float32)
    pairsum = oh0 + oh1

    @pl.when(i == 0)
    def _init():
        base_ref[...] = jnp.zeros_like(base_ref)
        ent_ref[0, 0] = 0.0

    rowi = lax.broadcasted_iota(jnp.int32, (BN, BN), 0)
    coli = lax.broadcasted_iota(jnp.int32, (BN, BN), 1)
    tri = (coli < rowi).astype(jnp.float32)
    prev = jnp.dot(tri, pairsum, preferred_element_type=jnp.float32)
    base_plus = base_ref[...] + prev
    rank0 = jnp.sum(oh0 * base_plus, axis=1, keepdims=True)
    rank1 = jnp.sum(oh1 * base_plus, axis=1, keepdims=True)

    idx0_ref[...] = i1
    idx1_ref[...] = i2
    rank0_ref[...] = rank0.astype(jnp.int32)
    rank1_ref[...] = rank1.astype(jnp.int32)
    w0_ref[...] = w0
    w1_ref[...] = 1.0 - w0

    base_ref[...] += jnp.sum(pairsum, axis=0, keepdims=True)
    lse = m1 + jnp.log(jnp.sum(jnp.exp(s - m1), axis=1, keepdims=True))
    logp = s - lse
    ent_ref[0, 0] += -jnp.sum(jnp.exp(logp) * logp)

    @pl.when(i == NB - 1)
    def _finalize():
        cnt_ref[...] = base_ref[...].astype(jnp.int32)
        er = lax.broadcasted_iota(jnp.int32, (E, E), 0)
        ec = lax.broadcasted_iota(jnp.int32, (E, E), 1)
        triu = (er < ec).astype(jnp.float32)
        off_ref[...] = jnp.dot(base_ref[...], triu,
                               preferred_element_type=jnp.float32,
                               precision=lax.Precision.HIGHEST
                               ).astype(jnp.int32)
        usage = base_ref[...] / N
        lb = jnp.mean((usage - 1.0 / E) ** 2)
        aux_ref[0, 0] = lb - 0.1 * (ent_ref[0, 0] / N)


def _run_gate(x, Wg, bg, *, BN):
    N, D = x.shape
    E = Wg.shape[1]
    NB = N // BN
    return pl.pallas_call(
        functools.partial(_gate_body, NB=NB, E=E, N=N),
        grid=(NB,),
        in_specs=[
            pl.BlockSpec((BN, D), lambda i: (i, 0)),
            pl.BlockSpec((D, E), lambda i: (0, 0)),
            pl.BlockSpec((1, E), lambda i: (0, 0)),
        ],
        out_specs=[pl.BlockSpec((BN, 1), lambda i: (i, 0))] * 6 + [
            pl.BlockSpec((1, E), lambda i: (0, 0)),
            pl.BlockSpec((1, E), lambda i: (0, 0)),
            pl.BlockSpec(memory_space=pltpu.SMEM),
        ],
        out_shape=[
            jax.ShapeDtypeStruct((N, 1), jnp.int32),
            jax.ShapeDtypeStruct((N, 1), jnp.int32),
            jax.ShapeDtypeStruct((N, 1), jnp.int32),
            jax.ShapeDtypeStruct((N, 1), jnp.int32),
            jax.ShapeDtypeStruct((N, 1), jnp.float32),
            jax.ShapeDtypeStruct((N, 1), jnp.float32),
            jax.ShapeDtypeStruct((1, E), jnp.int32),
            jax.ShapeDtypeStruct((1, E), jnp.int32),
            jax.ShapeDtypeStruct((1, 1), jnp.float32),
        ],
        scratch_shapes=[
            pltpu.VMEM((1, E), jnp.float32),
            pltpu.SMEM((1, 1), jnp.float32),
        ],
    )(x, Wg, bg.reshape(1, E))


def _sc_scatter(x, idx0, idx1, rank0, rank1, off):
    N, D = x.shape
    E = off.shape[0]
    M = 2 * N
    info = plsc.get_sparse_core_info()
    NC, NS = info.num_cores, info.num_subcores
    NW = NC * NS
    TPW = N // NW
    TB = 16
    NCH = TPW // TB
    mesh = plsc.VectorSubcoreMesh(core_axis_name="c", subcore_axis_name="s")

    @functools.partial(
        pl.kernel, mesh=mesh,
        compiler_params=pltpu.CompilerParams(needs_layout_passes=False),
        out_type=[
            jax.ShapeDtypeStruct((M, D), jnp.float32),
            jax.ShapeDtypeStruct((N // TB, TB), jnp.int32),
            jax.ShapeDtypeStruct((N // TB, TB), jnp.int32),
        ],
        scratch_types=[
            pltpu.VMEM((E,), jnp.int32),
            pltpu.VMEM((TPW,), jnp.int32),
            pltpu.VMEM((TPW,), jnp.int32),
            pltpu.VMEM((TPW,), jnp.int32),
            pltpu.VMEM((TPW,), jnp.int32),
            pltpu.VMEM((NCH, TB), jnp.int32),
            pltpu.VMEM((NCH, TB), jnp.int32),
            pltpu.VMEM((TB, D), jnp.float32),
            pltpu.VMEM((TB, D), jnp.float32),
            pltpu.SemaphoreType.DMA,
            pltpu.SemaphoreType.DMA,
            pltpu.SemaphoreType.DMA,
            pltpu.SemaphoreType.DMA,
            pltpu.SemaphoreType.DMA,
            pltpu.SemaphoreType.DMA,
        ],
    )
    def k(x_hbm, i0_hbm, i1_hbm, r0_hbm, r1_hbm, off_hbm,
          xs_hbm, d0_hbm, d1_hbm,
          off_v, i0v, i1v, r0v, r1v, d0all, d1all, xbuf0, xbuf1,
          rs0, rs1, w0s0, w0s1, w1s0, w1s1):
        wid = lax.axis_index("s") * NC + lax.axis_index("c")
        base = wid * TPW
        pltpu.sync_copy(off_hbm, off_v)
        pltpu.sync_copy(i0_hbm.at[pl.ds(base, TPW)], i0v)
        pltpu.sync_copy(i1_hbm.at[pl.ds(base, TPW)], i1v)
        pltpu.sync_copy(r0_hbm.at[pl.ds(base, TPW)], r0v)
        pltpu.sync_copy(r1_hbm.at[pl.ds(base, TPW)], r1v)
        for c in range(NCH):
            for j in range(TB // 16):
                src = pl.ds(c * TB + j * 16, 16)
                dst = pl.ds(j * 16, 16)
                d0all[c, dst] = plsc.load_gather(off_v, [i0v[src]]) + r0v[src]
                d1all[c, dst] = plsc.load_gather(off_v, [i1v[src]]) + r1v[src]
        pltpu.sync_copy(d0all, d0_hbm.at[pl.ds(wid * NCH, NCH)])
        pltpu.sync_copy(d1all, d1_hbm.at[pl.ds(wid * NCH, NCH)])

        xbuf = [xbuf0, xbuf1]
        rsem = [rs0, rs1]
        w0sem = [w0s0, w0s1]
        w1sem = [w1s0, w1s1]
        rh = [None, None]
        sc0 = [None, None]
        sc1 = [None, None]

        def read(c):
            return pltpu.async_copy(
                x_hbm.at[pl.ds(base + c * TB, TB)], xbuf[c & 1],
                rsem[c & 1])

        rh[0] = read(0)
        for c in range(NCH):
            b = c & 1
            rh[b].wait()
            sc0[b] = pltpu.async_copy(xbuf[b], xs_hbm.at[d0all.at[c]],
                                      w0sem[b])
            sc1[b] = pltpu.async_copy(xbuf[b], xs_hbm.at[d1all.at[c]],
                                      w1sem[b])
            if c + 1 < NCH:
                if c >= 1:
                    sc0[1 - b].wait()
                    sc1[1 - b].wait()
                rh[1 - b] = read(c + 1)
        sc0[0].wait()
        sc1[0].wait()
        sc0[1].wait()
        sc1[1].wait()

    xs, d0, d1 = k(x, idx0, idx1, rank0, rank1, off)
    return xs, d0.reshape(N), d1.reshape(N)


def _group_body(s_tile, s_exp, s_rows, xs_ref, W1_ref, b1_ref, g_ref,
                be_ref, W2_ref, b2_ref, ys_ref):
    i = pl.program_id(0)
    rs = s_rows[0, i]
    re_ = s_rows[1, i]

    @pl.when(re_ > rs)
    def _work():
        h = jnp.dot(xs_ref[...].astype(jnp.bfloat16),
                    W1_ref[0].astype(jnp.bfloat16),
                    preferred_element_type=jnp.float32) + b1_ref[0]
        h = 0.5 * h * (1.0 + lax.erf(h * jnp.float32(_SQRT_HALF)))
        mu = jnp.mean(h, axis=-1, keepdims=True)
        var = jnp.mean((h - mu) ** 2, axis=-1, keepdims=True)
        hn = (h - mu) / jnp.sqrt(var + 1e-5) * g_ref[0] + be_ref[0]
        y = jnp.dot(hn.astype(jnp.bfloat16), W2_ref[0].astype(jnp.bfloat16),
                    preferred_element_type=jnp.float32) + b2_ref[0]
        ri = lax.broadcasted_iota(jnp.int32, y.shape, 0)
        ys_ref[...] = jnp.where((ri >= rs) & (ri < re_), y, ys_ref[...])


def _run_group(xs, W1, b1, gamma, beta, W2, b2, off, *, BT):
    M, D = xs.shape
    E, _, H = W1.shape
    O = W2.shape[2]
    NT = M // BT
    NW = NT + E - 1

    tile_pts = jnp.arange(NT + 1, dtype=jnp.int32) * BT
    pts = jnp.sort(jnp.concatenate([tile_pts, off[1:]]))
    a, b = pts[:-1], pts[1:]
    s_tile = jnp.minimum(a // BT, NT - 1).astype(jnp.int32)
    s_exp = jnp.clip(jnp.searchsorted(off, a, side="right").astype(jnp.int32)
                     - 1, 0, E - 1)
    rs = a - s_tile * BT
    re_ = jnp.maximum(b - s_tile * BT, rs)
    s_rows = jnp.stack([rs, re_]).astype(jnp.int32)

    grid_spec = pltpu.PrefetchScalarGridSpec(
        num_scalar_prefetch=3,
        grid=(NW,),
        in_specs=[
            pl.BlockSpec((BT, D), lambda i, st, se, sr: (st[i], 0)),
            pl.BlockSpec((1, D, H), lambda i, st, se, sr: (se[i], 0, 0)),
            pl.BlockSpec((1, 1, H), lambda i, st, se, sr: (se[i], 0, 0)),
            pl.BlockSpec((1, 1, H), lambda i, st, se, sr: (se[i], 0, 0)),
            pl.BlockSpec((1, 1, H), lambda i, st, se, sr: (se[i], 0, 0)),
            pl.BlockSpec((1, H, O), lambda i, st, se, sr: (se[i], 0, 0)),
            pl.BlockSpec((1, 1, O), lambda i, st, se, sr: (se[i], 0, 0)),
        ],
        out_specs=pl.BlockSpec((BT, O), lambda i, st, se, sr: (st[i], 0)),
    )
    return pl.pallas_call(
        _group_body,
        grid_spec=grid_spec,
        out_shape=jax.ShapeDtypeStruct((M, O), jnp.float32),
    )(s_tile, s_exp, s_rows, xs, W1,
      b1.reshape(E, 1, H), gamma.reshape(E, 1, H), beta.reshape(E, 1, H),
      W2, b2.reshape(E, 1, O))


def _sc_combine(ys, d0, d1, w0, w1):
    M, O = ys.shape
    N = d0.shape[0]
    info = plsc.get_sparse_core_info()
    NC, NS = info.num_cores, info.num_subcores
    NW = NC * NS
    TPW = N // NW
    TC = 64
    mesh = plsc.VectorSubcoreMesh(core_axis_name="c", subcore_axis_name="s")

    @functools.partial(
        pl.kernel, mesh=mesh,
        compiler_params=pltpu.CompilerParams(needs_layout_passes=False),
        out_type=jax.ShapeDtypeStruct((N, O), jnp.float32),
        scratch_types=[
            pltpu.VMEM((TC,), jnp.int32),
            pltpu.VMEM((TC,), jnp.int32),
            pltpu.VMEM((TC,), jnp.float32),
            pltpu.VMEM((TC,), jnp.float32),
            pltpu.VMEM((TC, O), jnp.float32),
            pltpu.VMEM((TC, O), jnp.float32),
            pltpu.VMEM((TC, O), jnp.float32),
            pltpu.SemaphoreType.DMA,
            pltpu.SemaphoreType.DMA,
        ],
    )
    def k(ys_hbm, d0_hbm, d1_hbm, w0_hbm, w1_hbm, out_hbm,
          i0v, i1v, w0v, w1v, b0, b1, ob, sem0, sem1):
        wid = lax.axis_index("s") * NC + lax.axis_index("c")
        base = wid * TPW

        def chunk(c, carry):
            t0 = base + c * TC
            pltpu.sync_copy(d0_hbm.at[pl.ds(t0, TC)], i0v)
            pltpu.sync_copy(d1_hbm.at[pl.ds(t0, TC)], i1v)
            pltpu.sync_copy(w0_hbm.at[pl.ds(t0, TC)], w0v)
            pltpu.sync_copy(w1_hbm.at[pl.ds(t0, TC)], w1v)
            g0 = pltpu.async_copy(ys_hbm.at[i0v], b0, sem0)
            g1 = pltpu.async_copy(ys_hbm.at[i1v], b1, sem1)
            g0.wait()
            g1.wait()

            def token(n, tc):
                nv = jnp.full((16,), n, jnp.int32)
                wv0 = plsc.load_gather(w0v, [nv])
                wv1 = plsc.load_gather(w1v, [nv])
                for j in range(O // 16):
                    sl = pl.ds(j * 16, 16)
                    ob[n, sl] = b0[n, sl] * wv0 + b1[n, sl] * wv1
                return tc

            lax.fori_loop(0, TC, token, 0)
            pltpu.sync_copy(ob, out_hbm.at[pl.ds(t0, TC)])
            return carry

        lax.fori_loop(0, TPW // TC, chunk, 0)

    return k(ys, d0, d1, w0, w1)


@jax.jit
def kernel(x, Wg, bg, W1, b1, gamma, beta, W2, b2):
    N, D = x.shape
    E = Wg.shape[1]

    (idx0, idx1, rank0, rank1, w0, w1, cnt, off, aux) = _run_gate(
        x, Wg, bg, BN=512)

    off16 = off.reshape(E)
    xs, d0, d1 = _sc_scatter(
        x, idx0.reshape(N), idx1.reshape(N), rank0.reshape(N),
        rank1.reshape(N), off16)

    ys = _run_group(xs, W1, b1, gamma, beta, W2, b2, off16, BT=1024)

    out = _sc_combine(ys, d0, d1, w0.reshape(N), w1.reshape(N))
    return out, aux[0, 0]


# speedup vs baseline: 1.1333x; 1.0452x over previous
import functools

import jax
import jax.numpy as jnp
from jax import lax
from jax.experimental import pallas as pl
from jax.experimental.pallas import tpu as pltpu
from jax.experimental.pallas import tpu_sc as plsc

_SQRT_HALF = 0.7071067811865476


def _gate_body(x_ref, Wg_ref, bg_ref, idx0_ref, idx1_ref, rank0_ref,
               rank1_ref, w0_ref, w1_ref, cnt_ref, off_ref, aux_ref,
               base_ref, ent_ref, *, NB, E, N):
    i = pl.program_id(0)
    s = jnp.dot(x_ref[...], Wg_ref[...],
                preferred_element_type=jnp.float32) + bg_ref[...]
    BN = s.shape[0]
    iota = lax.broadcasted_iota(jnp.int32, s.shape, 1)
    m1 = jnp.max(s, axis=1, keepdims=True)
    i1 = jnp.min(jnp.where(s == m1, iota, E), axis=1, keepdims=True)
    s2 = jnp.where(iota == i1, -jnp.inf, s)
    m2 = jnp.max(s2, axis=1, keepdims=True)
    i2 = jnp.min(jnp.where(s2 == m2, iota, E), axis=1, keepdims=True)
    w0 = jax.nn.sigmoid(m1 - m2)
    oh0 = (iota == i1).astype(jnp.float32)
    oh1 = (iota == i2).astype(jnp.float32)
    pairsum = oh0 + oh1

    @pl.when(i == 0)
    def _init():
        base_ref[...] = jnp.zeros_like(base_ref)
        ent_ref[0, 0] = 0.0

    rowi = lax.broadcasted_iota(jnp.int32, (BN, BN), 0)
    coli = lax.broadcasted_iota(jnp.int32, (BN, BN), 1)
    tri = (coli < rowi).astype(jnp.float32)
    prev = jnp.dot(tri, pairsum, preferred_element_type=jnp.float32)
    base_plus = base_ref[...] + prev
    rank0 = jnp.sum(oh0 * base_plus, axis=1, keepdims=True)
    rank1 = jnp.sum(oh1 * base_plus, axis=1, keepdims=True)

    idx0_ref[...] = i1
    idx1_ref[...] = i2
    rank0_ref[...] = rank0.astype(jnp.int32)
    rank1_ref[...] = rank1.astype(jnp.int32)
    w0_ref[...] = w0
    w1_ref[...] = 1.0 - w0

    base_ref[...] += jnp.sum(pairsum, axis=0, keepdims=True)
    lse = m1 + jnp.log(jnp.sum(jnp.exp(s - m1), axis=1, keepdims=True))
    logp = s - lse
    ent_ref[0, 0] += -jnp.sum(jnp.exp(logp) * logp)

    @pl.when(i == NB - 1)
    def _finalize():
        cnt_ref[...] = base_ref[...].astype(jnp.int32)
        er = lax.broadcasted_iota(jnp.int32, (E, E), 0)
        ec = lax.broadcasted_iota(jnp.int32, (E, E), 1)
        triu = (er < ec).astype(jnp.float32)
        off_ref[...] = jnp.dot(base_ref[...], triu,
                               preferred_element_type=jnp.float32,
                               precision=lax.Precision.HIGHEST
                               ).astype(jnp.int32)
        usage = base_ref[...] / N
        lb = jnp.mean((usage - 1.0 / E) ** 2)
        aux_ref[0, 0] = lb - 0.1 * (ent_ref[0, 0] / N)


def _run_gate(x, Wg, bg, *, BN):
    N, D = x.shape
    E = Wg.shape[1]
    NB = N // BN
    return pl.pallas_call(
        functools.partial(_gate_body, NB=NB, E=E, N=N),
        grid=(NB,),
        in_specs=[
            pl.BlockSpec((BN, D), lambda i: (i, 0)),
            pl.BlockSpec((D, E), lambda i: (0, 0)),
            pl.BlockSpec((1, E), lambda i: (0, 0)),
        ],
        out_specs=[pl.BlockSpec((BN, 1), lambda i: (i, 0))] * 6 + [
            pl.BlockSpec((1, E), lambda i: (0, 0)),
            pl.BlockSpec((1, E), lambda i: (0, 0)),
            pl.BlockSpec(memory_space=pltpu.SMEM),
        ],
        out_shape=[
            jax.ShapeDtypeStruct((N, 1), jnp.int32),
            jax.ShapeDtypeStruct((N, 1), jnp.int32),
            jax.ShapeDtypeStruct((N, 1), jnp.int32),
            jax.ShapeDtypeStruct((N, 1), jnp.int32),
            jax.ShapeDtypeStruct((N, 1), jnp.float32),
            jax.ShapeDtypeStruct((N, 1), jnp.float32),
            jax.ShapeDtypeStruct((1, E), jnp.int32),
            jax.ShapeDtypeStruct((1, E), jnp.int32),
            jax.ShapeDtypeStruct((1, 1), jnp.float32),
        ],
        scratch_shapes=[
            pltpu.VMEM((1, E), jnp.float32),
            pltpu.SMEM((1, 1), jnp.float32),
        ],
    )(x, Wg, bg.reshape(1, E))


def _sc_scatter(x, idx0, idx1, rank0, rank1, off):
    N, D = x.shape
    E = off.shape[0]
    M = 2 * N
    info = plsc.get_sparse_core_info()
    NC, NS = info.num_cores, info.num_subcores
    NW = NC * NS
    TPW = N // NW
    TB = 16
    NCH = TPW // TB
    mesh = plsc.VectorSubcoreMesh(core_axis_name="c", subcore_axis_name="s")

    @functools.partial(
        pl.kernel, mesh=mesh,
        compiler_params=pltpu.CompilerParams(needs_layout_passes=False),
        out_type=[
            jax.ShapeDtypeStruct((M, D), jnp.float32),
            jax.ShapeDtypeStruct((N // TB, TB), jnp.int32),
            jax.ShapeDtypeStruct((N // TB, TB), jnp.int32),
        ],
        scratch_types=[
            pltpu.VMEM((E,), jnp.int32),
            pltpu.VMEM((TPW,), jnp.int32),
            pltpu.VMEM((TPW,), jnp.int32),
            pltpu.VMEM((TPW,), jnp.int32),
            pltpu.VMEM((TPW,), jnp.int32),
            pltpu.VMEM((NCH, TB), jnp.int32),
            pltpu.VMEM((NCH, TB), jnp.int32),
            pltpu.VMEM((TB, D), jnp.float32),
            pltpu.VMEM((TB, D), jnp.float32),
            pltpu.SemaphoreType.DMA,
            pltpu.SemaphoreType.DMA,
            pltpu.SemaphoreType.DMA,
            pltpu.SemaphoreType.DMA,
            pltpu.SemaphoreType.DMA,
            pltpu.SemaphoreType.DMA,
        ],
    )
    def k(x_hbm, i0_hbm, i1_hbm, r0_hbm, r1_hbm, off_hbm,
          xs_hbm, d0_hbm, d1_hbm,
          off_v, i0v, i1v, r0v, r1v, d0all, d1all, xbuf0, xbuf1,
          rs0, rs1, w0s0, w0s1, w1s0, w1s1):
        wid = lax.axis_index("s") * NC + lax.axis_index("c")
        base = wid * TPW
        pltpu.sync_copy(off_hbm, off_v)
        pltpu.sync_copy(i0_hbm.at[pl.ds(base, TPW)], i0v)
        pltpu.sync_copy(i1_hbm.at[pl.ds(base, TPW)], i1v)
        pltpu.sync_copy(r0_hbm.at[pl.ds(base, TPW)], r0v)
        pltpu.sync_copy(r1_hbm.at[pl.ds(base, TPW)], r1v)
        for c in range(NCH):
            for j in range(TB // 16):
                src = pl.ds(c * TB + j * 16, 16)
                dst = pl.ds(j * 16, 16)
                d0all[c, dst] = plsc.load_gather(off_v, [i0v[src]]) + r0v[src]
                d1all[c, dst] = plsc.load_gather(off_v, [i1v[src]]) + r1v[src]
        pltpu.sync_copy(d0all, d0_hbm.at[pl.ds(wid * NCH, NCH)])
        pltpu.sync_copy(d1all, d1_hbm.at[pl.ds(wid * NCH, NCH)])

        xbuf = [xbuf0, xbuf1]
        rsem = [rs0, rs1]
        w0sem = [w0s0, w0s1]
        w1sem = [w1s0, w1s1]
        rh = [None, None]
        sc0 = [None, None]
        sc1 = [None, None]

        def read(c):
            return pltpu.async_copy(
                x_hbm.at[pl.ds(base + c * TB, TB)], xbuf[c & 1],
                rsem[c & 1])

        rh[0] = read(0)
        for c in range(NCH):
            b = c & 1
            rh[b].wait()
            sc0[b] = pltpu.async_copy(xbuf[b], xs_hbm.at[d0all.at[c]],
                                      w0sem[b])
            sc1[b] = pltpu.async_copy(xbuf[b], xs_hbm.at[d1all.at[c]],
                                      w1sem[b])
            if c + 1 < NCH:
                if c >= 1:
                    sc0[1 - b].wait()
                    sc1[1 - b].wait()
                rh[1 - b] = read(c + 1)
        sc0[0].wait()
        sc1[0].wait()
        sc0[1].wait()
        sc1[1].wait()

    xs, d0, d1 = k(x, idx0, idx1, rank0, rank1, off)
    return xs, d0.reshape(N), d1.reshape(N)


def _group_body(s_tile, s_exp, s_rows, xs_ref, W1_ref, b1_ref, g_ref,
                be_ref, W2_ref, b2_ref, ys_ref):
    i = pl.program_id(0)
    rs = s_rows[0, i]
    re_ = s_rows[1, i]

    @pl.when(re_ > rs)
    def _work():
        h = jnp.dot(xs_ref[...].astype(jnp.bfloat16),
                    W1_ref[0].astype(jnp.bfloat16),
                    preferred_element_type=jnp.float32) + b1_ref[0]
        h = 0.5 * h * (1.0 + lax.erf(h * jnp.float32(_SQRT_HALF)))
        mu = jnp.mean(h, axis=-1, keepdims=True)
        var = jnp.mean((h - mu) ** 2, axis=-1, keepdims=True)
        hn = (h - mu) / jnp.sqrt(var + 1e-5) * g_ref[0] + be_ref[0]
        y = jnp.dot(hn.astype(jnp.bfloat16), W2_ref[0].astype(jnp.bfloat16),
                    preferred_element_type=jnp.float32) + b2_ref[0]
        ri = lax.broadcasted_iota(jnp.int32, y.shape, 0)
        ys_ref[...] = jnp.where((ri >= rs) & (ri < re_), y, ys_ref[...])


def _run_group(xs, W1, b1, gamma, beta, W2, b2, off, *, BT):
    M, D = xs.shape
    E, _, H = W1.shape
    O = W2.shape[2]
    NT = M // BT
    NW = NT + E - 1

    tile_pts = jnp.arange(NT + 1, dtype=jnp.int32) * BT
    pts = jnp.sort(jnp.concatenate([tile_pts, off[1:]]))
    a, b = pts[:-1], pts[1:]
    s_tile = jnp.minimum(a // BT, NT - 1).astype(jnp.int32)
    s_exp = jnp.clip(jnp.searchsorted(off, a, side="right").astype(jnp.int32)
                     - 1, 0, E - 1)
    rs = a - s_tile * BT
    re_ = jnp.maximum(b - s_tile * BT, rs)
    s_rows = jnp.stack([rs, re_]).astype(jnp.int32)

    grid_spec = pltpu.PrefetchScalarGridSpec(
        num_scalar_prefetch=3,
        grid=(NW,),
        in_specs=[
            pl.BlockSpec((BT, D), lambda i, st, se, sr: (st[i], 0)),
            pl.BlockSpec((1, D, H), lambda i, st, se, sr: (se[i], 0, 0)),
            pl.BlockSpec((1, 1, H), lambda i, st, se, sr: (se[i], 0, 0)),
            pl.BlockSpec((1, 1, H), lambda i, st, se, sr: (se[i], 0, 0)),
            pl.BlockSpec((1, 1, H), lambda i, st, se, sr: (se[i], 0, 0)),
            pl.BlockSpec((1, H, O), lambda i, st, se, sr: (se[i], 0, 0)),
            pl.BlockSpec((1, 1, O), lambda i, st, se, sr: (se[i], 0, 0)),
        ],
        out_specs=pl.BlockSpec((BT, O), lambda i, st, se, sr: (st[i], 0)),
    )
    return pl.pallas_call(
        _group_body,
        grid_spec=grid_spec,
        out_shape=jax.ShapeDtypeStruct((M, O), jnp.float32),
    )(s_tile, s_exp, s_rows, xs, W1,
      b1.reshape(E, 1, H), gamma.reshape(E, 1, H), beta.reshape(E, 1, H),
      W2, b2.reshape(E, 1, O))


def _sc_combine(ys, d0, d1, w0, w1):
    M, O = ys.shape
    N = d0.shape[0]
    info = plsc.get_sparse_core_info()
    NC, NS = info.num_cores, info.num_subcores
    NW = NC * NS
    TPW = N // NW
    TC = 32
    NCH = TPW // TC
    mesh = plsc.VectorSubcoreMesh(core_axis_name="c", subcore_axis_name="s")

    @functools.partial(
        pl.kernel, mesh=mesh,
        compiler_params=pltpu.CompilerParams(needs_layout_passes=False),
        out_type=jax.ShapeDtypeStruct((N, O), jnp.float32),
        scratch_types=[
            pltpu.VMEM((NCH, TC), jnp.int32),
            pltpu.VMEM((NCH, TC), jnp.int32),
            pltpu.VMEM((TPW,), jnp.float32),
            pltpu.VMEM((TPW,), jnp.float32),
            pltpu.VMEM((TC, O), jnp.float32),
            pltpu.VMEM((TC, O), jnp.float32),
            pltpu.VMEM((TC, O), jnp.float32),
            pltpu.VMEM((TC, O), jnp.float32),
            pltpu.VMEM((TC, O), jnp.float32),
            pltpu.SemaphoreType.DMA,
            pltpu.SemaphoreType.DMA,
            pltpu.SemaphoreType.DMA,
            pltpu.SemaphoreType.DMA,
        ],
    )
    def k(ys_hbm, d0_hbm, d1_hbm, w0_hbm, w1_hbm, out_hbm,
          i0all, i1all, w0v, w1v, b0a, b0b, b1a, b1b, ob,
          g0sa, g0sb, g1sa, g1sb):
        wid = lax.axis_index("s") * NC + lax.axis_index("c")
        base = wid * TPW
        pltpu.sync_copy(d0_hbm.at[pl.ds(wid * NCH, NCH)], i0all)
        pltpu.sync_copy(d1_hbm.at[pl.ds(wid * NCH, NCH)], i1all)
        pltpu.sync_copy(w0_hbm.at[pl.ds(base, TPW)], w0v)
        pltpu.sync_copy(w1_hbm.at[pl.ds(base, TPW)], w1v)

        b0 = [b0a, b0b]
        b1 = [b1a, b1b]
        g0sem = [g0sa, g0sb]
        g1sem = [g1sa, g1sb]
        gh = [None, None]

        def gathers(c):
            b = c & 1
            h0 = pltpu.async_copy(ys_hbm.at[i0all.at[c]], b0[b], g0sem[b])
            h1 = pltpu.async_copy(ys_hbm.at[i1all.at[c]], b1[b], g1sem[b])
            return (h0, h1)

        gh[0] = gathers(0)
        for c in range(NCH):
            b = c & 1
            gh[b][0].wait()
            gh[b][1].wait()
            if c + 1 < NCH:
                gh[1 - b] = gathers(c + 1)

            def token(n, tc):
                nv = jnp.full((16,), c * TC, jnp.int32) + n
                wv0 = plsc.load_gather(w0v, [nv])
                wv1 = plsc.load_gather(w1v, [nv])
                for j in range(O // 16):
                    sl = pl.ds(j * 16, 16)
                    ob[n, sl] = b0[b][n, sl] * wv0 + b1[b][n, sl] * wv1
                return tc

            lax.fori_loop(0, TC, token, 0)
            pltpu.sync_copy(ob, out_hbm.at[pl.ds(base + c * TC, TC)])

    return k(ys, d0.reshape(NW * NCH, TC), d1.reshape(NW * NCH, TC), w0, w1)


@jax.jit
def kernel(x, Wg, bg, W1, b1, gamma, beta, W2, b2):
    N, D = x.shape
    E = Wg.shape[1]

    (idx0, idx1, rank0, rank1, w0, w1, cnt, off, aux) = _run_gate(
        x, Wg, bg, BN=512)

    off16 = off.reshape(E)
    xs, d0, d1 = _sc_scatter(
        x, idx0.reshape(N), idx1.reshape(N), rank0.reshape(N),
        rank1.reshape(N), off16)

    ys = _run_group(xs, W1, b1, gamma, beta, W2, b2, off16, BT=1024)

    out = _sc_combine(ys, d0, d1, w0.reshape(N), w1.reshape(N))
    return out, aux[0, 0]
